# Initial kernel scaffold; baseline (speedup 1.0000x reference)
import jax
import jax.numpy as jnp
from jax.experimental import pallas as pl


def kernel(tensor):
    raise NotImplementedError("write your pallas kernel here")


# speedup vs baseline: 26.2833x; 26.2833x over previous
import jax
import jax.numpy as jnp
import numpy as np
from jax.experimental import pallas as pl
from jax.experimental.pallas import tpu as pltpu

_SHAPE = (128, 32768)
_N = _SHAPE[0] * _SHAPE[1]
_LOW_RANK = 4026530
_LOW_W = np.float32(0.25)
_HIGH_W = np.float32(0.75)
_MASK31 = np.int32(0x7FFFFFFF)
_INT_MIN = np.int32(-(2**31))
_INT_MAX = np.int32(2**31 - 1)

_ROWS_PER_BLK = 8
_NBLK = _SHAPE[0] // _ROWS_PER_BLK


def _key_to_f32(k):
    b = k ^ (jax.lax.shift_right_arithmetic(k, 31) & _MASK31)
    return jax.lax.bitcast_convert_type(b, jnp.float32)


def _search_body(x_ref, t_ref, m_ref, scr_ref):
    i = pl.program_id(0)

    @pl.when(i < _NBLK)
    def _load():
        x = x_ref[...]
        b = jax.lax.bitcast_convert_type(x, jnp.int32)
        keys = b ^ (jax.lax.shift_right_arithmetic(b, 31) & _MASK31)
        scr_ref[pl.ds(i * _ROWS_PER_BLK, _ROWS_PER_BLK), :] = keys

    @pl.when(i < _NBLK)
    def _init():
        t_ref[0, 0] = jnp.float32(0.0)
        m_ref[0, 0] = jnp.float32(0.0)

    @pl.when(i == _NBLK)
    def _search():
        def count_lt(q):
            def chunk(j, acc):
                c = scr_ref[pl.ds(j * _ROWS_PER_BLK, _ROWS_PER_BLK), :]
                return acc + jnp.sum((c < q).astype(jnp.int32))
            return jax.lax.fori_loop(0, _NBLK, chunk, jnp.int32(0))

        def step(s, p):
            bit = jnp.left_shift(np.int32(1), (31 - s).astype(jnp.int32))
            q = p + bit
            c = count_lt(q)
            return jnp.where(c <= _LOW_RANK, q, p)

        p = jax.lax.fori_loop(0, 32, step, _INT_MIN)

        def succ_chunk(j, carry):
            c_le, mn_above = carry
            c = scr_ref[pl.ds(j * _ROWS_PER_BLK, _ROWS_PER_BLK), :]
            c_le = c_le + jnp.sum((c <= p).astype(jnp.int32))
            above = jnp.where(c > p, c, _INT_MAX)
            return c_le, jnp.minimum(mn_above, jnp.min(above))

        c_le, mn_above = jax.lax.fori_loop(
            0, _NBLK, succ_chunk, (jnp.int32(0), _INT_MAX))
        p_high = jnp.where(c_le >= _LOW_RANK + 2, p, mn_above)

        v_low = _key_to_f32(p)
        v_high = _key_to_f32(p_high)
        tresh = v_low * _LOW_W + v_high * _HIGH_W
        t_ref[0, 0] = tresh
        m_ref[0, 0] = jnp.where(tresh >= v_high, v_high, v_low)


def _apply_body(t_ref, m_ref, x_ref, o_ref):
    tresh = t_ref[0, 0]
    m = m_ref[0, 0]
    x = x_ref[...]
    o_ref[...] = jnp.where(x > tresh, m, x)


@jax.jit
def kernel(tensor):
    tresh, m = pl.pallas_call(
        _search_body,
        grid=(_NBLK + 1,),
        in_specs=[pl.BlockSpec(
            (_ROWS_PER_BLK, _SHAPE[1]),
            lambda i: (jnp.minimum(i, _NBLK - 1), 0))],
        out_specs=[pl.BlockSpec(memory_space=pltpu.SMEM),
                   pl.BlockSpec(memory_space=pltpu.SMEM)],
        out_shape=[jax.ShapeDtypeStruct((1, 1), jnp.float32),
                   jax.ShapeDtypeStruct((1, 1), jnp.float32)],
        scratch_shapes=[pltpu.VMEM(_SHAPE, jnp.int32)],
    )(tensor)

    return pl.pallas_call(
        _apply_body,
        grid=(_NBLK,),
        in_specs=[pl.BlockSpec(memory_space=pltpu.SMEM),
                  pl.BlockSpec(memory_space=pltpu.SMEM),
                  pl.BlockSpec((_ROWS_PER_BLK, _SHAPE[1]),
                               lambda i: (i, 0))],
        out_specs=pl.BlockSpec((_ROWS_PER_BLK, _SHAPE[1]), lambda i: (i, 0)),
        out_shape=jax.ShapeDtypeStruct(_SHAPE, jnp.float32),
    )(tresh, m, tensor)


# speedup vs baseline: 27.5148x; 1.0469x over previous
import jax
import jax.numpy as jnp
import numpy as np
from jax.experimental import pallas as pl
from jax.experimental.pallas import tpu as pltpu

_SHAPE = (128, 32768)
_N = _SHAPE[0] * _SHAPE[1]
_LOW_RANK = 4026530
_LOW_W = np.float32(0.25)
_HIGH_W = np.float32(0.75)
_MASK31 = np.int32(0x7FFFFFFF)
_INT_MIN = np.int32(-(2**31))
_INT_MAX = np.int32(2**31 - 1)

_ROWS_PER_BLK = 8
_NBLK = _SHAPE[0] // _ROWS_PER_BLK


def _key_to_f32(k):
    b = k ^ (jax.lax.shift_right_arithmetic(k, 31) & _MASK31)
    return jax.lax.bitcast_convert_type(b, jnp.float32)


def _body(x_ref, o_ref, scr_ref, tm_ref):
    i = pl.program_id(0)

    @pl.when(i < _NBLK)
    def _load():
        x = x_ref[...]
        b = jax.lax.bitcast_convert_type(x, jnp.int32)
        keys = b ^ (jax.lax.shift_right_arithmetic(b, 31) & _MASK31)
        scr_ref[pl.ds(i * _ROWS_PER_BLK, _ROWS_PER_BLK), :] = keys

    @pl.when(i == _NBLK)
    def _search():
        def count_lt(q):
            def chunk(j, acc):
                c = scr_ref[pl.ds(j * _ROWS_PER_BLK, _ROWS_PER_BLK), :]
                return acc + jnp.sum((c < q).astype(jnp.int32))
            return jax.lax.fori_loop(0, _NBLK, chunk, jnp.int32(0))

        def step(s, p):
            bit = jnp.left_shift(np.int32(1), (31 - s).astype(jnp.int32))
            q = p + bit
            c = count_lt(q)
            return jnp.where(c <= _LOW_RANK, q, p)

        p = jax.lax.fori_loop(0, 32, step, _INT_MIN)

        def succ_chunk(j, carry):
            c_le, mn_above = carry
            c = scr_ref[pl.ds(j * _ROWS_PER_BLK, _ROWS_PER_BLK), :]
            c_le = c_le + jnp.sum((c <= p).astype(jnp.int32))
            above = jnp.where(c > p, c, _INT_MAX)
            return c_le, jnp.minimum(mn_above, jnp.min(above))

        c_le, mn_above = jax.lax.fori_loop(
            0, _NBLK, succ_chunk, (jnp.int32(0), _INT_MAX))
        p_high = jnp.where(c_le >= _LOW_RANK + 2, p, mn_above)

        v_low = _key_to_f32(p)
        v_high = _key_to_f32(p_high)
        tresh = v_low * _LOW_W + v_high * _HIGH_W
        tm_ref[0] = tresh
        tm_ref[1] = jnp.where(tresh >= v_high, v_high, v_low)

    @pl.when(i == _NBLK + 1)
    def _apply():
        tresh = tm_ref[0]
        m = tm_ref[1]

        def chunk(j, carry):
            keys = scr_ref[pl.ds(j * _ROWS_PER_BLK, _ROWS_PER_BLK), :]
            x = _key_to_f32(keys)
            o_ref[pl.ds(j * _ROWS_PER_BLK, _ROWS_PER_BLK), :] = (
                jnp.where(x > tresh, m, x))
            return carry

        jax.lax.fori_loop(0, _NBLK, chunk, jnp.int32(0))


@jax.jit
def kernel(tensor):
    return pl.pallas_call(
        _body,
        grid=(_NBLK + 2,),
        in_specs=[pl.BlockSpec(
            (_ROWS_PER_BLK, _SHAPE[1]),
            lambda i: (jnp.minimum(i, _NBLK - 1), 0))],
        out_specs=pl.BlockSpec(_SHAPE, lambda i: (0, 0)),
        out_shape=jax.ShapeDtypeStruct(_SHAPE, jnp.float32),
        scratch_shapes=[pltpu.VMEM(_SHAPE, jnp.int32),
                        pltpu.SMEM((2,), jnp.float32)],
    )(tensor)


# speedup vs baseline: 33.6469x; 1.2229x over previous
import jax
import jax.numpy as jnp
import numpy as np
from jax.experimental import pallas as pl
from jax.experimental.pallas import tpu as pltpu

_SHAPE = (128, 32768)
_N = _SHAPE[0] * _SHAPE[1]
_LOW_RANK = 4026530
_LOW_W = np.float32(0.25)
_HIGH_W = np.float32(0.75)
_MASK31 = np.int32(0x7FFFFFFF)
_INT_MIN = np.int32(-(2**31))
_INT_MAX = np.int32(2**31 - 1)

_ROWS_PER_BLK = 8
_NBLK = _SHAPE[0] // _ROWS_PER_BLK


def _key_to_f32(k):
    b = k ^ (jax.lax.shift_right_arithmetic(k, 31) & _MASK31)
    return jax.lax.bitcast_convert_type(b, jnp.float32)


def _body(x_ref, o_ref, scr_ref, tm_ref):
    i = pl.program_id(0)

    @pl.when(i < _NBLK)
    def _load():
        x = x_ref[...]
        b = jax.lax.bitcast_convert_type(x, jnp.int32)
        keys = b ^ (jax.lax.shift_right_arithmetic(b, 31) & _MASK31)
        scr_ref[pl.ds(i * _ROWS_PER_BLK, _ROWS_PER_BLK), :] = keys

    @pl.when(i == _NBLK)
    def _search():
        def count_lt(q):
            def chunk(j, acc):
                c = scr_ref[pl.ds(j * _ROWS_PER_BLK, _ROWS_PER_BLK), :]
                m = (c < q).astype(jnp.int32)
                return acc + m.reshape(64, 4, 8, 128).sum(axis=0)
            acc = jax.lax.fori_loop(
                0, _NBLK, chunk, jnp.zeros((4, 8, 128), jnp.int32))
            return jnp.sum(acc)

        def step(s, p):
            bit = jnp.left_shift(np.int32(1), (31 - s).astype(jnp.int32))
            q = p + bit
            c = count_lt(q)
            return jnp.where(c <= _LOW_RANK, q, p)

        p = jax.lax.fori_loop(0, 32, step, _INT_MIN)

        def succ_chunk(j, carry):
            c_le, mn_above = carry
            c = scr_ref[pl.ds(j * _ROWS_PER_BLK, _ROWS_PER_BLK), :]
            c_le = c_le + jnp.sum((c <= p).astype(jnp.int32))
            above = jnp.where(c > p, c, _INT_MAX)
            return c_le, jnp.minimum(mn_above, jnp.min(above))

        c_le, mn_above = jax.lax.fori_loop(
            0, _NBLK, succ_chunk, (jnp.int32(0), _INT_MAX))
        p_high = jnp.where(c_le >= _LOW_RANK + 2, p, mn_above)

        v_low = _key_to_f32(p)
        v_high = _key_to_f32(p_high)
        tresh = v_low * _LOW_W + v_high * _HIGH_W
        tm_ref[0] = tresh
        tm_ref[1] = jnp.where(tresh >= v_high, v_high, v_low)

    @pl.when(i == _NBLK + 1)
    def _apply():
        tresh = tm_ref[0]
        m = tm_ref[1]

        def chunk(j, carry):
            keys = scr_ref[pl.ds(j * _ROWS_PER_BLK, _ROWS_PER_BLK), :]
            x = _key_to_f32(keys)
            o_ref[pl.ds(j * _ROWS_PER_BLK, _ROWS_PER_BLK), :] = (
                jnp.where(x > tresh, m, x))
            return carry

        jax.lax.fori_loop(0, _NBLK, chunk, jnp.int32(0))


@jax.jit
def kernel(tensor):
    return pl.pallas_call(
        _body,
        grid=(_NBLK + 2,),
        in_specs=[pl.BlockSpec(
            (_ROWS_PER_BLK, _SHAPE[1]),
            lambda i: (jnp.minimum(i, _NBLK - 1), 0))],
        out_specs=pl.BlockSpec(_SHAPE, lambda i: (0, 0)),
        out_shape=jax.ShapeDtypeStruct(_SHAPE, jnp.float32),
        scratch_shapes=[pltpu.VMEM(_SHAPE, jnp.int32),
                        pltpu.SMEM((2,), jnp.float32)],
    )(tensor)
